# baseline (speedup 1.0000x reference)
import jax
import jax.numpy as jnp
from jax import lax
from jax.experimental import pallas as pl
from jax.experimental.pallas import tpu as pltpu
from jax.experimental.pallas import tpu_sc as plsc

BATCH = 16384
N_FIELDS = 26
VOCAB = 1000000
VOCABP = 1000448
NW = 32
BPW = BATCH // NW
BPH = BPW // 2
IDX_PH = BPH * N_FIELDS
NGROUP = BPH // 16


def _emb_body(xt_hbm, tt_hbm, out_hbm, idx1_v, vals_v, o_v, sem, gsem):
    wid = lax.axis_index("s") * 2 + lax.axis_index("c")
    base = wid * BPW
    tflat = tt_hbm.at[0]

    def stage(h):
        hofs = pl.multiple_of(h * IDX_PH, IDX_PH)
        for f in range(N_FIELDS):
            dst = idx1_v.at[pl.ds(hofs + f * BPH, BPH)]
            src = xt_hbm.at[f].at[pl.ds(base + h * BPH, BPH)]
            pltpu.make_async_copy(src, dst, sem).start()
        for f in range(N_FIELDS):
            dst = idx1_v.at[pl.ds(hofs + f * BPH, BPH)]
            src = xt_hbm.at[f].at[pl.ds(base + h * BPH, BPH)]
            pltpu.make_async_copy(src, dst, sem).wait()

    def gather(h, s):
        hofs = pl.multiple_of(h * IDX_PH, IDX_PH)
        return pltpu.make_async_copy(
            tflat.at[idx1_v.at[pl.ds(hofs, IDX_PH)]],
            vals_v.at[pl.ds(hofs, IDX_PH)], s)

    def reduce_half(h):
        hofs = pl.multiple_of(h * IDX_PH, IDX_PH)
        oofs = pl.multiple_of(h * BPH, BPH)

        def group(g, carry):
            o16 = hofs + pl.multiple_of(g * 16, 16)
            acc0 = vals_v[pl.ds(o16, 16)]
            acc1 = vals_v[pl.ds(o16 + BPH, 16)]
            for f in range(2, N_FIELDS, 2):
                acc0 = acc0 + vals_v[pl.ds(o16 + f * BPH, 16)]
                acc1 = acc1 + vals_v[pl.ds(o16 + (f + 1) * BPH, 16)]
            s = acc0 + acc1
            o_v[pl.ds(oofs + pl.multiple_of(g * 16, 16), 16)] = (
                1.0 / (1.0 + jnp.exp(-s)))
            return carry

        lax.fori_loop(0, NGROUP, group, 0)

    stage(0)
    g0 = gather(0, gsem)
    g0.start()
    stage(1)
    g1 = gather(1, sem)
    g1.start()
    g0.wait()
    reduce_half(0)
    g1.wait()
    reduce_half(1)
    pltpu.sync_copy(o_v, out_hbm.at[pl.ds(base, BPW)])


def _emb_call(xt, tt):
    mesh = plsc.VectorSubcoreMesh(core_axis_name="c", subcore_axis_name="s")
    return pl.kernel(
        _emb_body,
        out_type=jax.ShapeDtypeStruct((BATCH,), jnp.float32),
        mesh=mesh,
        scratch_types=[
            pltpu.VMEM((2 * IDX_PH,), jnp.int32),
            pltpu.VMEM((2 * IDX_PH,), jnp.float32),
            pltpu.VMEM((BPW,), jnp.float32),
            pltpu.SemaphoreType.DMA,
            pltpu.SemaphoreType.DMA,
        ],
        compiler_params=pltpu.CompilerParams(
            needs_layout_passes=False, use_tc_tiling_on_sc=False),
    )(xt, tt)


def kernel(x, table):
    xt = x.astype(jnp.int32).T
    tp = jnp.pad(table.T, ((0, 0), (0, VOCABP - VOCAB)))
    return _emb_call(xt, tp).reshape(BATCH, 1)


# speedup vs baseline: 1.2478x; 1.2478x over previous
import jax
import jax.numpy as jnp
from jax import lax
from jax.experimental import pallas as pl
from jax.experimental.pallas import tpu as pltpu
from jax.experimental.pallas import tpu_sc as plsc

BATCH = 16384
N_FIELDS = 26
VOCAB = 1000000
VOCABP = 1000448
NW = 32
BPW = BATCH // NW
IDX_PW = BPW * N_FIELDS
NGROUP = BPW // 16


def _emb_body(xt_hbm, tt_hbm, out_hbm, idx1_v, vals_v, o_v, sem):
    wid = lax.axis_index("s") * 2 + lax.axis_index("c")
    base = wid * BPW
    for f in range(N_FIELDS):
        dst = idx1_v.at[pl.ds(pl.multiple_of(f * BPW, BPW), BPW)]
        pltpu.make_async_copy(xt_hbm.at[f].at[pl.ds(base, BPW)], dst, sem).start()
    for f in range(N_FIELDS):
        dst = idx1_v.at[pl.ds(pl.multiple_of(f * BPW, BPW), BPW)]
        pltpu.make_async_copy(xt_hbm.at[f].at[pl.ds(base, BPW)], dst, sem).wait()

    tflat = tt_hbm.at[0]
    pltpu.make_async_copy(tflat.at[idx1_v], vals_v, sem).start()
    pltpu.make_async_copy(tflat.at[idx1_v], vals_v, sem).wait()

    def group(g, carry):
        o16 = pl.multiple_of(g * 16, 16)
        acc0 = vals_v[pl.ds(o16, 16)]
        acc1 = vals_v[pl.ds(o16 + BPW, 16)]
        for f in range(2, N_FIELDS, 2):
            acc0 = acc0 + vals_v[pl.ds(o16 + f * BPW, 16)]
            acc1 = acc1 + vals_v[pl.ds(o16 + (f + 1) * BPW, 16)]
        s = acc0 + acc1
        o_v[pl.ds(o16, 16)] = 1.0 / (1.0 + jnp.exp(-s))
        return carry

    lax.fori_loop(0, NGROUP, group, 0)
    pltpu.sync_copy(o_v, out_hbm.at[pl.ds(base, BPW)])


def _emb_call(xt, tt):
    mesh = plsc.VectorSubcoreMesh(core_axis_name="c", subcore_axis_name="s")
    return pl.kernel(
        _emb_body,
        out_type=jax.ShapeDtypeStruct((BATCH,), jnp.float32),
        mesh=mesh,
        scratch_types=[
            pltpu.VMEM((IDX_PW,), jnp.int32),
            pltpu.VMEM((IDX_PW,), jnp.float32),
            pltpu.VMEM((BPW,), jnp.float32),
            pltpu.SemaphoreType.DMA,
        ],
        compiler_params=pltpu.CompilerParams(
            needs_layout_passes=False, use_tc_tiling_on_sc=True),
    )(xt, tt)


def kernel(x, table):
    xt = x.astype(jnp.int32).T
    return _emb_call(xt, table.T).reshape(BATCH, 1)
